# baseline (speedup 1.0000x reference)
import functools

import jax
import jax.numpy as jnp
from jax import lax
from jax.experimental import pallas as pl
from jax.experimental.pallas import tpu as pltpu
from jax.experimental.pallas import tpu_sc as plsc

HIDDEN_DIM = 4096
NUM_EXPERTS = 64
TOP_K = 8
LANES = 16
NUM_SC = 2
NUM_TEC = 16
NUM_WORKERS = NUM_SC * NUM_TEC
NEG_CAP = -3e38


def _logits_body(x_ref, w_ref, out_ref):
    out_ref[...] = lax.dot_general(
        x_ref[...], w_ref[...], (((1,), (1,)), ((), ())),
        preferred_element_type=jnp.float32)


def _compute_logits(x2d, W, block_t=512):
    T = x2d.shape[0]
    return pl.pallas_call(
        _logits_body,
        grid=(T // block_t,),
        in_specs=[
            pl.BlockSpec((block_t, HIDDEN_DIM), lambda i: (i, 0)),
            pl.BlockSpec((NUM_EXPERTS, HIDDEN_DIM), lambda i: (0, 0)),
        ],
        out_specs=pl.BlockSpec((block_t, NUM_EXPERTS), lambda i: (i, 0)),
        out_shape=jax.ShapeDtypeStruct((T, NUM_EXPERTS), jnp.float32),
    )(x2d, W)


def _make_topk(T):
    C = T // NUM_WORKERS
    G = C // LANES
    mesh = plsc.VectorSubcoreMesh(
        core_axis_name="c", subcore_axis_name="s",
        num_cores=NUM_SC, num_subcores=NUM_TEC)

    @functools.partial(
        pl.kernel,
        out_type=[
            jax.ShapeDtypeStruct((T * TOP_K,), jnp.float32),
            jax.ShapeDtypeStruct((T * TOP_K,), jnp.int32),
        ],
        mesh=mesh,
        compiler_params=pltpu.CompilerParams(needs_layout_passes=False),
        scratch_types=[
            pltpu.VMEM((C * NUM_EXPERTS,), jnp.float32),
            pltpu.VMEM((C * TOP_K,), jnp.float32),
            pltpu.VMEM((C * TOP_K,), jnp.int32),
        ],
    )
    def topk(logits_hbm, w_hbm, i_hbm, lg_v, wout_v, iout_v):
        wid = lax.axis_index("s") * NUM_SC + lax.axis_index("c")
        base = wid * C
        pltpu.sync_copy(
            logits_hbm.at[pl.ds(base * NUM_EXPERTS, C * NUM_EXPERTS)], lg_v)
        lanes = lax.iota(jnp.int32, LANES)

        def group(g, carry):
            rows = g * LANES + lanes
            row_off = rows * NUM_EXPERTS
            vals = []
            idxs = []
            for k in range(TOP_K):
                m = plsc.load_gather(lg_v, [row_off])
                mi = jnp.zeros((LANES,), jnp.int32)
                for e in range(1, NUM_EXPERTS):
                    v = plsc.load_gather(lg_v, [row_off + e])
                    c = v > m
                    m = jnp.where(c, v, m)
                    mi = jnp.where(c, jnp.full((LANES,), e, jnp.int32), mi)
                vals.append(m)
                idxs.append(mi)
                if k + 1 < TOP_K:
                    plsc.store_scatter(
                        lg_v, [row_off + mi],
                        jnp.full((LANES,), NEG_CAP, jnp.float32))
            mx = vals[0]
            exps = [jnp.exp(v - mx) for v in vals]
            s = exps[0]
            for t in exps[1:]:
                s = s + t
            inv = jnp.float32(1.0) / s
            out_off = rows * TOP_K
            for k in range(TOP_K):
                plsc.store_scatter(wout_v, [out_off + k], exps[k] * inv)
                plsc.store_scatter(iout_v, [out_off + k], idxs[k])
            return carry

        lax.fori_loop(0, G, group, 0)
        pltpu.sync_copy(wout_v, w_hbm.at[pl.ds(base * TOP_K, C * TOP_K)])
        pltpu.sync_copy(iout_v, i_hbm.at[pl.ds(base * TOP_K, C * TOP_K)])

    return topk


def kernel(x, W):
    B, S, D = x.shape
    T = B * S
    x2d = x.reshape(T, D)
    logits = _compute_logits(x2d, W)
    w, i = _make_topk(T)(logits.reshape(T * NUM_EXPERTS))
    return w.reshape(B, S, TOP_K), i.reshape(B, S, TOP_K)


# speedup vs baseline: 1.6109x; 1.6109x over previous
import functools

import jax
import jax.numpy as jnp
from jax import lax
from jax.experimental import pallas as pl
from jax.experimental.pallas import tpu as pltpu
from jax.experimental.pallas import tpu_sc as plsc

HIDDEN_DIM = 4096
NUM_EXPERTS = 64
TOP_K = 8
LANES = 16
NUM_SC = 2
NUM_TEC = 16
NUM_WORKERS = NUM_SC * NUM_TEC
NEG_CAP = -3e38


def _logits_body(x_ref, w_ref, out_ref):
    out_ref[...] = lax.dot_general(
        x_ref[...], w_ref[...], (((1,), (1,)), ((), ())),
        preferred_element_type=jnp.float32)


def _compute_logits(x2d, W, block_t=512):
    T = x2d.shape[0]
    return pl.pallas_call(
        _logits_body,
        grid=(T // block_t,),
        in_specs=[
            pl.BlockSpec((block_t, HIDDEN_DIM), lambda i: (i, 0)),
            pl.BlockSpec((NUM_EXPERTS, HIDDEN_DIM), lambda i: (0, 0)),
        ],
        out_specs=pl.BlockSpec((block_t, NUM_EXPERTS), lambda i: (i, 0)),
        out_shape=jax.ShapeDtypeStruct((T, NUM_EXPERTS), jnp.float32),
    )(x2d, W)


def _make_topk(T):
    C = T // NUM_WORKERS
    G = C // LANES
    mesh = plsc.VectorSubcoreMesh(
        core_axis_name="c", subcore_axis_name="s",
        num_cores=NUM_SC, num_subcores=NUM_TEC)

    @functools.partial(
        pl.kernel,
        out_type=[
            jax.ShapeDtypeStruct((T * TOP_K,), jnp.float32),
            jax.ShapeDtypeStruct((T * TOP_K,), jnp.int32),
        ],
        mesh=mesh,
        compiler_params=pltpu.CompilerParams(needs_layout_passes=False),
        scratch_types=[
            pltpu.VMEM((C * NUM_EXPERTS,), jnp.float32),
            pltpu.VMEM((C * TOP_K,), jnp.float32),
            pltpu.VMEM((C * TOP_K,), jnp.int32),
        ],
    )
    def topk(logits_hbm, w_hbm, i_hbm, lg_v, wout_v, iout_v):
        wid = lax.axis_index("s") * NUM_SC + lax.axis_index("c")
        base = wid * C
        pltpu.sync_copy(
            logits_hbm.at[pl.ds(base * NUM_EXPERTS, C * NUM_EXPERTS)], lg_v)
        lanes = lax.iota(jnp.int32, LANES)

        neg_vec = jnp.full((LANES,), NEG_CAP, jnp.float32)
        NG = NUM_EXPERTS // 8

        def group(g, carry):
            rows = g * LANES + lanes
            row_off = rows * NUM_EXPERTS
            gm = []
            gi = []
            for j in range(NG):
                m = plsc.load_gather(lg_v, [row_off + 8 * j])
                mi = jnp.full((LANES,), 8 * j, jnp.int32)
                for t in range(1, 8):
                    e = 8 * j + t
                    v = plsc.load_gather(lg_v, [row_off + e])
                    c = v > m
                    m = jnp.where(c, v, m)
                    mi = jnp.where(c, jnp.full((LANES,), e, jnp.int32), mi)
                gm.append(m)
                gi.append(mi)
            vals = []
            idxs = []
            for k in range(TOP_K):
                bm = gm[0]
                bi = gi[0]
                for j in range(1, NG):
                    c = gm[j] > bm
                    bm = jnp.where(c, gm[j], bm)
                    bi = jnp.where(c, gi[j], bi)
                vals.append(bm)
                idxs.append(bi)
                if k + 1 < TOP_K:
                    plsc.store_scatter(lg_v, [row_off + bi], neg_vec)
                    g8 = bi & jnp.full((LANES,), -8, jnp.int32)
                    goff = row_off + g8
                    nm = plsc.load_gather(lg_v, [goff])
                    nmi = g8
                    for t in range(1, 8):
                        v = plsc.load_gather(lg_v, [goff + t])
                        c = v > nm
                        nm = jnp.where(c, v, nm)
                        nmi = jnp.where(c, g8 + t, nmi)
                    bg = lax.shift_right_logical(bi, 3)
                    for j in range(NG):
                        cj = bg == j
                        gm[j] = jnp.where(cj, nm, gm[j])
                        gi[j] = jnp.where(cj, nmi, gi[j])
            mx = vals[0]
            exps = [jnp.exp(v - mx) for v in vals]
            s = exps[0]
            for t in exps[1:]:
                s = s + t
            inv = jnp.float32(1.0) / s
            out_off = rows * TOP_K
            for k in range(TOP_K):
                plsc.store_scatter(wout_v, [out_off + k], exps[k] * inv)
                plsc.store_scatter(iout_v, [out_off + k], idxs[k])
            return carry

        lax.fori_loop(0, G, group, 0)
        pltpu.sync_copy(wout_v, w_hbm.at[pl.ds(base * TOP_K, C * TOP_K)])
        pltpu.sync_copy(iout_v, i_hbm.at[pl.ds(base * TOP_K, C * TOP_K)])

    return topk


def kernel(x, W):
    B, S, D = x.shape
    T = B * S
    x2d = x.reshape(T, D)
    logits = _compute_logits(x2d, W)
    w, i = _make_topk(T)(logits.reshape(T * NUM_EXPERTS))
    return w.reshape(B, S, TOP_K), i.reshape(B, S, TOP_K)


# speedup vs baseline: 1.7973x; 1.1157x over previous
import functools

import jax
import jax.numpy as jnp
from jax import lax
from jax.experimental import pallas as pl
from jax.experimental.pallas import tpu as pltpu
from jax.experimental.pallas import tpu_sc as plsc

HIDDEN_DIM = 4096
NUM_EXPERTS = 64
TOP_K = 8
LANES = 16
NUM_SC = 2
NUM_TEC = 16
NUM_WORKERS = NUM_SC * NUM_TEC
NEG_CAP = -3e38


def _logits_body(x_ref, w_ref, out_ref):
    out_ref[...] = lax.dot_general(
        x_ref[...], w_ref[...], (((1,), (1,)), ((), ())),
        preferred_element_type=jnp.float32)


def _compute_logits_chunk(x2d, W, chunk, chunk_t, block_t=512):
    nblk = chunk_t // block_t
    blk0 = chunk * nblk
    return pl.pallas_call(
        _logits_body,
        grid=(nblk,),
        in_specs=[
            pl.BlockSpec((block_t, HIDDEN_DIM), lambda i: (blk0 + i, 0)),
            pl.BlockSpec((NUM_EXPERTS, HIDDEN_DIM), lambda i: (0, 0)),
        ],
        out_specs=pl.BlockSpec((block_t, NUM_EXPERTS), lambda i: (i, 0)),
        out_shape=jax.ShapeDtypeStruct((chunk_t, NUM_EXPERTS), jnp.float32),
    )(x2d, W)


def _make_topk(T):
    C = T // NUM_WORKERS
    G = C // LANES
    mesh = plsc.VectorSubcoreMesh(
        core_axis_name="c", subcore_axis_name="s",
        num_cores=NUM_SC, num_subcores=NUM_TEC)

    @functools.partial(
        pl.kernel,
        out_type=[
            jax.ShapeDtypeStruct((T * TOP_K,), jnp.float32),
            jax.ShapeDtypeStruct((T * TOP_K,), jnp.int32),
        ],
        mesh=mesh,
        compiler_params=pltpu.CompilerParams(needs_layout_passes=False),
        scratch_types=[
            pltpu.VMEM((C * NUM_EXPERTS,), jnp.float32),
            pltpu.VMEM((C * TOP_K,), jnp.float32),
            pltpu.VMEM((C * TOP_K,), jnp.int32),
        ],
    )
    def topk(logits_hbm, w_hbm, i_hbm, lg_v, wout_v, iout_v):
        wid = lax.axis_index("s") * NUM_SC + lax.axis_index("c")
        base = wid * C
        pltpu.sync_copy(
            logits_hbm.at[pl.ds(base * NUM_EXPERTS, C * NUM_EXPERTS)], lg_v)
        lanes = lax.iota(jnp.int32, LANES)

        neg_vec = jnp.full((LANES,), NEG_CAP, jnp.float32)
        NG = NUM_EXPERTS // 8

        def group(g, carry):
            rows = g * LANES + lanes
            row_off = rows * NUM_EXPERTS
            gm = []
            gi = []
            for j in range(NG):
                m = plsc.load_gather(lg_v, [row_off + 8 * j])
                mi = jnp.full((LANES,), 8 * j, jnp.int32)
                for t in range(1, 8):
                    e = 8 * j + t
                    v = plsc.load_gather(lg_v, [row_off + e])
                    c = v > m
                    m = jnp.where(c, v, m)
                    mi = jnp.where(c, jnp.full((LANES,), e, jnp.int32), mi)
                gm.append(m)
                gi.append(mi)
            vals = []
            idxs = []
            for k in range(TOP_K):
                bm = gm[0]
                bi = gi[0]
                for j in range(1, NG):
                    c = gm[j] > bm
                    bm = jnp.where(c, gm[j], bm)
                    bi = jnp.where(c, gi[j], bi)
                vals.append(bm)
                idxs.append(bi)
                if k + 1 < TOP_K:
                    plsc.store_scatter(lg_v, [row_off + bi], neg_vec)
                    g8 = bi & jnp.full((LANES,), -8, jnp.int32)
                    goff = row_off + g8
                    nm = plsc.load_gather(lg_v, [goff])
                    nmi = g8
                    for t in range(1, 8):
                        v = plsc.load_gather(lg_v, [goff + t])
                        c = v > nm
                        nm = jnp.where(c, v, nm)
                        nmi = jnp.where(c, g8 + t, nmi)
                    bg = lax.shift_right_logical(bi, 3)
                    for j in range(NG):
                        cj = bg == j
                        gm[j] = jnp.where(cj, nm, gm[j])
                        gi[j] = jnp.where(cj, nmi, gi[j])
            mx = vals[0]
            exps = [jnp.exp(v - mx) for v in vals]
            s = exps[0]
            for t in exps[1:]:
                s = s + t
            inv = jnp.float32(1.0) / s
            out_off = rows * TOP_K
            for k in range(TOP_K):
                plsc.store_scatter(wout_v, [out_off + k], exps[k] * inv)
                plsc.store_scatter(iout_v, [out_off + k], idxs[k])
            return carry

        lax.fori_loop(0, G, group, 0)
        pltpu.sync_copy(wout_v, w_hbm.at[pl.ds(base * TOP_K, C * TOP_K)])
        pltpu.sync_copy(iout_v, i_hbm.at[pl.ds(base * TOP_K, C * TOP_K)])

    return topk


def kernel(x, W):
    B, S, D = x.shape
    T = B * S
    x2d = x.reshape(T, D)
    nchunk = 4
    chunk_t = T // nchunk
    topk = _make_topk(chunk_t)
    ws = []
    idxs = []
    for ci in range(nchunk):
        lg = _compute_logits_chunk(x2d, W, ci, chunk_t)
        w, i = topk(lg.reshape(chunk_t * NUM_EXPERTS))
        ws.append(w)
        idxs.append(i)
    w = jnp.concatenate(ws).reshape(B, S, TOP_K)
    i = jnp.concatenate(idxs).reshape(B, S, TOP_K)
    return w, i


# speedup vs baseline: 1.7986x; 1.0007x over previous
import functools

import jax
import jax.numpy as jnp
from jax import lax
from jax.experimental import pallas as pl
from jax.experimental.pallas import tpu as pltpu
from jax.experimental.pallas import tpu_sc as plsc

HIDDEN_DIM = 4096
NUM_EXPERTS = 64
TOP_K = 8
LANES = 16
NUM_SC = 2
NUM_TEC = 16
NUM_WORKERS = NUM_SC * NUM_TEC
NEG_CAP = -3e38


def _logits_body(xa_ref, xb_ref, wa_ref, wb_ref, out_ref):
    dn = (((1,), (1,)), ((), ()))
    out_ref[...] = (
        lax.dot_general(xa_ref[...], wa_ref[...], dn,
                        preferred_element_type=jnp.float32)
        + lax.dot_general(xb_ref[...], wb_ref[...], dn,
                          preferred_element_type=jnp.float32))


def _compute_logits_chunk(x2d, W, chunk, chunk_t, block_t=512):
    nblk = chunk_t // block_t
    blk0 = chunk * nblk
    half = HIDDEN_DIM // 2
    return pl.pallas_call(
        _logits_body,
        grid=(nblk,),
        in_specs=[
            pl.BlockSpec((block_t, half), lambda i: (blk0 + i, 0)),
            pl.BlockSpec((block_t, half), lambda i: (blk0 + i, 1)),
            pl.BlockSpec((NUM_EXPERTS, half), lambda i: (0, 0)),
            pl.BlockSpec((NUM_EXPERTS, half), lambda i: (0, 1)),
        ],
        out_specs=pl.BlockSpec((block_t, NUM_EXPERTS), lambda i: (i, 0)),
        out_shape=jax.ShapeDtypeStruct((chunk_t, NUM_EXPERTS), jnp.float32),
    )(x2d, x2d, W, W)


def _make_topk(T):
    C = T // NUM_WORKERS
    G = C // LANES
    mesh = plsc.VectorSubcoreMesh(
        core_axis_name="c", subcore_axis_name="s",
        num_cores=NUM_SC, num_subcores=NUM_TEC)

    @functools.partial(
        pl.kernel,
        out_type=[
            jax.ShapeDtypeStruct((T * TOP_K,), jnp.float32),
            jax.ShapeDtypeStruct((T * TOP_K,), jnp.int32),
        ],
        mesh=mesh,
        compiler_params=pltpu.CompilerParams(needs_layout_passes=False),
        scratch_types=[
            pltpu.VMEM((C * NUM_EXPERTS,), jnp.float32),
            pltpu.VMEM((C * TOP_K,), jnp.float32),
            pltpu.VMEM((C * TOP_K,), jnp.int32),
        ],
    )
    def topk(logits_hbm, w_hbm, i_hbm, lg_v, wout_v, iout_v):
        wid = lax.axis_index("s") * NUM_SC + lax.axis_index("c")
        base = wid * C
        pltpu.sync_copy(
            logits_hbm.at[pl.ds(base * NUM_EXPERTS, C * NUM_EXPERTS)], lg_v)
        lanes = lax.iota(jnp.int32, LANES)

        neg_vec = jnp.full((LANES,), NEG_CAP, jnp.float32)
        NG = NUM_EXPERTS // 8

        def group(g, carry):
            rows = g * LANES + lanes
            row_off = rows * NUM_EXPERTS
            gm = []
            gi = []
            for j in range(NG):
                m = plsc.load_gather(lg_v, [row_off + 8 * j])
                mi = jnp.full((LANES,), 8 * j, jnp.int32)
                for t in range(1, 8):
                    e = 8 * j + t
                    v = plsc.load_gather(lg_v, [row_off + e])
                    c = v > m
                    m = jnp.where(c, v, m)
                    mi = jnp.where(c, jnp.full((LANES,), e, jnp.int32), mi)
                gm.append(m)
                gi.append(mi)
            vals = []
            idxs = []
            for k in range(TOP_K):
                bm = gm[0]
                bi = gi[0]
                for j in range(1, NG):
                    c = gm[j] > bm
                    bm = jnp.where(c, gm[j], bm)
                    bi = jnp.where(c, gi[j], bi)
                vals.append(bm)
                idxs.append(bi)
                if k + 1 < TOP_K:
                    plsc.store_scatter(lg_v, [row_off + bi], neg_vec)
                    g8 = bi & jnp.full((LANES,), -8, jnp.int32)
                    goff = row_off + g8
                    nm = plsc.load_gather(lg_v, [goff])
                    nmi = g8
                    for t in range(1, 8):
                        v = plsc.load_gather(lg_v, [goff + t])
                        c = v > nm
                        nm = jnp.where(c, v, nm)
                        nmi = jnp.where(c, g8 + t, nmi)
                    bg = lax.shift_right_logical(bi, 3)
                    for j in range(NG):
                        cj = bg == j
                        gm[j] = jnp.where(cj, nm, gm[j])
                        gi[j] = jnp.where(cj, nmi, gi[j])
            mx = vals[0]
            exps = [jnp.exp(v - mx) for v in vals]
            s = exps[0]
            for t in exps[1:]:
                s = s + t
            inv = jnp.float32(1.0) / s
            out_off = rows * TOP_K
            for k in range(TOP_K):
                plsc.store_scatter(wout_v, [out_off + k], exps[k] * inv)
                plsc.store_scatter(iout_v, [out_off + k], idxs[k])
            return carry

        lax.fori_loop(0, G, group, 0)
        pltpu.sync_copy(wout_v, w_hbm.at[pl.ds(base * TOP_K, C * TOP_K)])
        pltpu.sync_copy(iout_v, i_hbm.at[pl.ds(base * TOP_K, C * TOP_K)])

    return topk


def kernel(x, W):
    B, S, D = x.shape
    T = B * S
    x2d = x.reshape(T, D)
    nchunk = 4
    chunk_t = T // nchunk
    topk = _make_topk(chunk_t)
    ws = []
    idxs = []
    for ci in range(nchunk):
        lg = _compute_logits_chunk(x2d, W, ci, chunk_t)
        w, i = topk(lg.reshape(chunk_t * NUM_EXPERTS))
        ws.append(w)
        idxs.append(i)
    w = jnp.concatenate(ws).reshape(B, S, TOP_K)
    i = jnp.concatenate(idxs).reshape(B, S, TOP_K)
    return w, i
